# Initial kernel scaffold; baseline (speedup 1.0000x reference)
import jax
import jax.numpy as jnp
from jax.experimental import pallas as pl


def kernel(u_row, i_col, user_table, item_table):
    raise NotImplementedError("write your pallas kernel here")


# speedup vs baseline: 107.1842x; 107.1842x over previous
import functools

import jax
import jax.numpy as jnp
from jax import lax
from jax.experimental import pallas as pl
from jax.experimental.pallas import tpu as pltpu
from jax.experimental.pallas import tpu_sc as plsc

NC = 2
NS = 16
D = 16


def _mesh():
    return plsc.VectorSubcoreMesh(core_axis_name="c", subcore_axis_name="s")


def _make_deg_kernel(E, NP, W):
    EPT = E // NS
    RPT = NP // NS
    ZR = 368

    @functools.partial(
        pl.kernel,
        out_type=(jax.ShapeDtypeStruct((NP, D), jnp.float32),
                  jax.ShapeDtypeStruct((NP, D), jnp.float32)),
        mesh=_mesh(),
        compiler_params=pltpu.CompilerParams(use_tc_tiling_on_sc=False),
        scratch_types=[
            pltpu.VMEM((W,), jnp.int32),
            pltpu.VMEM((W, D), jnp.float32),
            pltpu.VMEM((ZR, D), jnp.float32),
            pltpu.VMEM_SHARED((NP, D), jnp.float32),
        ],
    )
    def deg_kernel(u_hbm, i_hbm, degu_hbm, degi_hbm,
                   idx_v, ones_v, zero_v, acc_s):
        c = lax.axis_index("c")
        s = lax.axis_index("s")

        def fill_ones(i, _):
            ones_v[i, :] = jnp.ones((D,), jnp.float32)
            return 0

        lax.fori_loop(0, W, fill_ones, 0)

        def fill_zero(i, _):
            zero_v[i, :] = jnp.zeros((D,), jnp.float32)
            return 0

        lax.fori_loop(0, ZR, fill_zero, 0)
        base_r = s * RPT

        def run(idx_hbm, deg_hbm):
            def zero_acc(i, _):
                pltpu.sync_copy(zero_v, acc_s.at[pl.ds(base_r + i * ZR, ZR)])
                return 0

            lax.fori_loop(0, RPT // ZR, zero_acc, 0)
            plsc.subcore_barrier()
            base = s * EPT

            def body(i, _):
                pltpu.sync_copy(idx_hbm.at[pl.ds(base + i * W, W)], idx_v)
                pltpu.sync_copy(ones_v, acc_s.at[idx_v], add=True)
                return 0

            lax.fori_loop(0, EPT // W, body, 0)
            plsc.subcore_barrier()

            def copy_out(i, _):
                pltpu.sync_copy(acc_s.at[pl.ds(base_r + i * ZR, ZR)], zero_v)
                pltpu.sync_copy(zero_v, deg_hbm.at[pl.ds(base_r + i * ZR, ZR)])
                return 0

            lax.fori_loop(0, RPT // ZR, copy_out, 0)

        @pl.when(c == 0)
        def _():
            run(u_hbm, degu_hbm)

        @pl.when(c == 1)
        def _():
            run(i_hbm, degi_hbm)

    return deg_kernel


def _make_agg_kernel(E, NP, W):
    EPT = E // NS
    RPT = NP // NS
    ZR = 368

    @functools.partial(
        pl.kernel,
        out_type=(jax.ShapeDtypeStruct((NP, D), jnp.float32),
                  jax.ShapeDtypeStruct((NP, D), jnp.float32)),
        mesh=_mesh(),
        compiler_params=pltpu.CompilerParams(use_tc_tiling_on_sc=False),
        scratch_types=[
            pltpu.VMEM((W,), jnp.int32),
            pltpu.VMEM((W,), jnp.int32),
            pltpu.VMEM((W, D), jnp.float32),
            pltpu.VMEM((ZR, D), jnp.float32),
            pltpu.VMEM_SHARED((NP, D), jnp.float32),
            pltpu.SemaphoreType.DMA,
        ],
    )
    def agg_kernel(u_hbm, i_hbm, us_hbm, is_hbm, outu_hbm, outi_hbm,
                   dst_v, src_v, rows_v, zero_v, acc_s, sem):
        c = lax.axis_index("c")
        s = lax.axis_index("s")

        def fill_zero(i, _):
            zero_v[i, :] = jnp.zeros((D,), jnp.float32)
            return 0

        lax.fori_loop(0, ZR, fill_zero, 0)
        base_r = s * RPT

        def run(dst_hbm, src_hbm, tab_hbm, out_hbm):
            def zero_acc(i, _):
                pltpu.sync_copy(zero_v, acc_s.at[pl.ds(base_r + i * ZR, ZR)])
                return 0

            lax.fori_loop(0, RPT // ZR, zero_acc, 0)
            plsc.subcore_barrier()
            base = s * EPT

            def body(i, _):
                off = base + i * W
                pltpu.sync_copy(dst_hbm.at[pl.ds(off, W)], dst_v)
                pltpu.sync_copy(src_hbm.at[pl.ds(off, W)], src_v)
                pltpu.async_copy(tab_hbm.at[src_v], rows_v, sem).wait()
                pltpu.sync_copy(rows_v, acc_s.at[dst_v], add=True)
                return 0

            lax.fori_loop(0, EPT // W, body, 0)
            plsc.subcore_barrier()

            def copy_out(i, _):
                pltpu.sync_copy(acc_s.at[pl.ds(base_r + i * ZR, ZR)], zero_v)
                pltpu.sync_copy(zero_v, out_hbm.at[pl.ds(base_r + i * ZR, ZR)])
                return 0

            lax.fori_loop(0, RPT // ZR, copy_out, 0)

        @pl.when(c == 0)
        def _():
            run(u_hbm, i_hbm, is_hbm, outu_hbm)

        @pl.when(c == 1)
        def _():
            run(i_hbm, u_hbm, us_hbm, outi_hbm)

    return agg_kernel


def _scale_rows(x, deg):
    M = x.shape[0]
    R = 2000
    grid = (M // R,) if M % R == 0 else (M // R + 1,)

    def body(x_ref, d_ref, o_ref):
        d = d_ref[...]
        dis = jnp.where(d > 0, lax.rsqrt(d), 0.0)
        o_ref[...] = x_ref[...] * dis

    return pl.pallas_call(
        body,
        out_shape=jax.ShapeDtypeStruct((M, D), jnp.float32),
        grid=grid,
        in_specs=[
            pl.BlockSpec((R, D), lambda m: (m, 0)),
            pl.BlockSpec((R, 1), lambda m: (m, 0)),
        ],
        out_specs=pl.BlockSpec((R, D), lambda m: (m, 0)),
    )(x, deg)


def kernel(u_row, i_col, user_table, item_table):
    E = u_row.shape[0]
    N = user_table.shape[0]
    NP = ((N + NS * 8 - 1) // (NS * 8)) * NS * 8
    W = 1000

    deg_u, deg_i = _make_deg_kernel(E, NP, W)(u_row, i_col)
    du = deg_u[:N, :1]
    di = deg_i[:N, :1]

    user_s = _scale_rows(user_table, du)
    item_s = _scale_rows(item_table, di)

    out_u, out_i = _make_agg_kernel(E, NP, W)(u_row, i_col, user_s, item_s)

    user_emb = _scale_rows(out_u[:N], du)
    item_emb = _scale_rows(out_i[:N], di)
    return user_emb, item_emb


# speedup vs baseline: 128.8936x; 1.2025x over previous
import functools

import jax
import jax.numpy as jnp
from jax import lax
from jax.experimental import pallas as pl
from jax.experimental.pallas import tpu as pltpu
from jax.experimental.pallas import tpu_sc as plsc

NC = 2
NS = 16
D = 16


def _mesh():
    return plsc.VectorSubcoreMesh(core_axis_name="c", subcore_axis_name="s")


def _rsqrt_rows(d):
    i = plsc.bitcast(d, jnp.int32)
    y = plsc.bitcast(jnp.full((16,), 0x5F3759DF, jnp.int32) - (i >> 1),
                     jnp.float32)
    for _ in range(3):
        y = y * (1.5 - 0.5 * d * y * y)
    return jnp.where(d > 0.0, y, 0.0)


def _make_dis_kernel(E, N, W, ZR):
    EPT = E // NS
    RPT = N // NS

    @functools.partial(
        pl.kernel,
        out_type=(jax.ShapeDtypeStruct((N, D), jnp.float32),
                  jax.ShapeDtypeStruct((N, D), jnp.float32),
                  jax.ShapeDtypeStruct((N, D), jnp.float32),
                  jax.ShapeDtypeStruct((N, D), jnp.float32)),
        mesh=_mesh(),
        compiler_params=pltpu.CompilerParams(use_tc_tiling_on_sc=False, needs_layout_passes=False),
        scratch_types=[
            pltpu.VMEM((W,), jnp.int32),
            pltpu.VMEM((W, D), jnp.float32),
            pltpu.VMEM((ZR, D), jnp.float32),
            pltpu.VMEM_SHARED((N, D), jnp.float32),
        ],
    )
    def dis_kernel(u_hbm, i_hbm, ut_hbm, it_hbm,
                   disi_hbm, items_hbm, disu_hbm, users_hbm,
                   idx_v, ones_v, chunk_v, acc_s):
        c = lax.axis_index("c")
        s = lax.axis_index("s")
        base_r = s * RPT

        def fill_zero(i, _):
            chunk_v[i, :] = jnp.zeros((D,), jnp.float32)
            return 0

        lax.fori_loop(0, ZR, fill_zero, 0)

        def zero_acc(i, _):
            pltpu.sync_copy(chunk_v, acc_s.at[pl.ds(base_r + i * ZR, ZR)])
            return 0

        lax.fori_loop(0, RPT // ZR, zero_acc, 0)

        def fill_ones(i, _):
            ones_v[i, :] = jnp.ones((D,), jnp.float32)
            return 0

        lax.fori_loop(0, W, fill_ones, 0)
        plsc.subcore_barrier()

        def run(idx_hbm, tab_hbm, dis_hbm, tabs_hbm):
            base = s * EPT

            def body(i, _):
                pltpu.sync_copy(idx_hbm.at[pl.ds(base + i * W, W)], idx_v)
                pltpu.sync_copy(ones_v, acc_s.at[idx_v], add=True)
                return 0

            lax.fori_loop(0, EPT // W, body, 0)
            plsc.subcore_barrier()

            tabc = ones_v.at[pl.ds(0, ZR)]

            def epilogue(k, _):
                r0 = base_r + k * ZR
                pltpu.sync_copy(acc_s.at[pl.ds(r0, ZR)], chunk_v)

                def dis_row(i, _):
                    chunk_v[i, :] = _rsqrt_rows(chunk_v[i, :])
                    return 0

                lax.fori_loop(0, ZR, dis_row, 0)
                pltpu.sync_copy(chunk_v, dis_hbm.at[pl.ds(r0, ZR)])
                pltpu.sync_copy(tab_hbm.at[pl.ds(r0, ZR)], tabc)

                def scale_row(i, _):
                    tabc[i, :] = tabc[i, :] * chunk_v[i, :]
                    return 0

                lax.fori_loop(0, ZR, scale_row, 0)
                pltpu.sync_copy(tabc, tabs_hbm.at[pl.ds(r0, ZR)])
                return 0

            lax.fori_loop(0, RPT // ZR, epilogue, 0)

        @pl.when(c == 0)
        def _():
            run(i_hbm, it_hbm, disi_hbm, items_hbm)

        @pl.when(c == 1)
        def _():
            run(u_hbm, ut_hbm, disu_hbm, users_hbm)

    return dis_kernel


def _make_agg_kernel(E, N, W, ZR):
    EPT = E // NS
    RPT = N // NS

    @functools.partial(
        pl.kernel,
        out_type=(jax.ShapeDtypeStruct((N, D), jnp.float32),
                  jax.ShapeDtypeStruct((N, D), jnp.float32)),
        mesh=_mesh(),
        compiler_params=pltpu.CompilerParams(use_tc_tiling_on_sc=False, needs_layout_passes=False),
        scratch_types=[
            pltpu.VMEM((W,), jnp.int32),
            pltpu.VMEM((W,), jnp.int32),
            pltpu.VMEM((W, D), jnp.float32),
            pltpu.VMEM((ZR, D), jnp.float32),
            pltpu.VMEM_SHARED((N, D), jnp.float32),
            pltpu.SemaphoreType.DMA,
        ],
    )
    def agg_kernel(u_hbm, i_hbm, users_hbm, items_hbm, disu_hbm, disi_hbm,
                   outu_hbm, outi_hbm,
                   dst_v, src_v, rows_v, zero_v, acc_s, sem):
        c = lax.axis_index("c")
        s = lax.axis_index("s")
        base_r = s * RPT

        def fill_zero(i, _):
            zero_v[i, :] = jnp.zeros((D,), jnp.float32)
            return 0

        lax.fori_loop(0, ZR, fill_zero, 0)

        def zero_acc(i, _):
            pltpu.sync_copy(zero_v, acc_s.at[pl.ds(base_r + i * ZR, ZR)])
            return 0

        lax.fori_loop(0, RPT // ZR, zero_acc, 0)
        plsc.subcore_barrier()

        def run(dst_hbm, src_hbm, tab_hbm, dis_hbm, out_hbm):
            base = s * EPT

            def body(i, _):
                off = base + i * W
                pltpu.sync_copy(dst_hbm.at[pl.ds(off, W)], dst_v)
                pltpu.sync_copy(src_hbm.at[pl.ds(off, W)], src_v)
                pltpu.async_copy(tab_hbm.at[src_v], rows_v, sem).wait()
                pltpu.sync_copy(rows_v, acc_s.at[dst_v], add=True)
                return 0

            lax.fori_loop(0, EPT // W, body, 0)
            plsc.subcore_barrier()

            disc = rows_v.at[pl.ds(0, ZR)]

            def epilogue(k, _):
                r0 = base_r + k * ZR
                pltpu.sync_copy(acc_s.at[pl.ds(r0, ZR)], zero_v)
                pltpu.sync_copy(dis_hbm.at[pl.ds(r0, ZR)], disc)

                def scale_row(i, _):
                    zero_v[i, :] = zero_v[i, :] * disc[i, :]
                    return 0

                lax.fori_loop(0, ZR, scale_row, 0)
                pltpu.sync_copy(zero_v, out_hbm.at[pl.ds(r0, ZR)])
                return 0

            lax.fori_loop(0, RPT // ZR, epilogue, 0)

        @pl.when(c == 0)
        def _():
            run(u_hbm, i_hbm, items_hbm, disu_hbm, outu_hbm)

        @pl.when(c == 1)
        def _():
            run(i_hbm, u_hbm, users_hbm, disi_hbm, outi_hbm)

    return agg_kernel


def kernel(u_row, i_col, user_table, item_table):
    E = u_row.shape[0]
    N = user_table.shape[0]
    W = 800
    ZR = 250

    dis_i, item_s, dis_u, user_s = _make_dis_kernel(E, N, W, ZR)(
        u_row, i_col, user_table, item_table)
    user_emb, item_emb = _make_agg_kernel(E, N, W, ZR)(
        u_row, i_col, user_s, item_s, dis_u, dis_i)
    return user_emb, item_emb


# speedup vs baseline: 142.7648x; 1.1076x over previous
import functools

import jax
import jax.numpy as jnp
from jax import lax
from jax.experimental import pallas as pl
from jax.experimental.pallas import tpu as pltpu
from jax.experimental.pallas import tpu_sc as plsc

NC = 2
NS = 16
D = 16

_SC_PARAMS = pltpu.CompilerParams(use_tc_tiling_on_sc=False,
                                  needs_layout_passes=False)


def _mesh():
    return plsc.VectorSubcoreMesh(core_axis_name="c", subcore_axis_name="s")


def _rsqrt_rows(d):
    i = plsc.bitcast(d, jnp.int32)
    y = plsc.bitcast(jnp.full((16,), 0x5F3759DF, jnp.int32) - (i >> 1),
                     jnp.float32)
    for _ in range(2):
        y = y * (1.5 - 0.5 * d * y * y)
    return jnp.where(d > 0.0, y, 0.0)


def _make_dis_kernel(E, N, W, ZR):
    EPT = E // NS
    RPT = N // NS
    NW = EPT // W
    assert NW % 2 == 0 and EPT % W == 0 and RPT % ZR == 0

    @functools.partial(
        pl.kernel,
        out_type=(jax.ShapeDtypeStruct((N, D), jnp.float32),
                  jax.ShapeDtypeStruct((N, D), jnp.float32),
                  jax.ShapeDtypeStruct((N, D), jnp.float32),
                  jax.ShapeDtypeStruct((N, D), jnp.float32)),
        mesh=_mesh(),
        compiler_params=_SC_PARAMS,
        scratch_types=[
            pltpu.VMEM((W,), jnp.int32),
            pltpu.VMEM((W,), jnp.int32),
            pltpu.VMEM((W, D), jnp.float32),
            pltpu.VMEM((ZR, D), jnp.float32),
            pltpu.VMEM_SHARED((N, D), jnp.float32),
            pltpu.SemaphoreType.DMA,
            pltpu.SemaphoreType.DMA,
        ],
    )
    def dis_kernel(u_hbm, i_hbm, ut_hbm, it_hbm,
                   disi_hbm, items_hbm, disu_hbm, users_hbm,
                   idx0_v, idx1_v, ones_v, chunk_v, acc_s, sem0, sem1):
        c = lax.axis_index("c")
        s = lax.axis_index("s")
        base_r = s * RPT

        def fill_zero(i, _):
            chunk_v[i, :] = jnp.zeros((D,), jnp.float32)
            return 0

        lax.fori_loop(0, ZR, fill_zero, 0, unroll=8)

        def zero_acc(i, _):
            pltpu.sync_copy(chunk_v, acc_s.at[pl.ds(base_r + i * ZR, ZR)])
            return 0

        lax.fori_loop(0, RPT // ZR, zero_acc, 0)

        def fill_ones(i, _):
            ones_v[i, :] = jnp.ones((D,), jnp.float32)
            return 0

        lax.fori_loop(0, W, fill_ones, 0, unroll=8)
        plsc.subcore_barrier()

        def run(idx_hbm, tab_hbm, dis_hbm, tabs_hbm):
            base = s * EPT

            def load_idx(w, buf):
                pltpu.sync_copy(idx_hbm.at[pl.ds(base + w * W, W)], buf)

            def scat(buf, sem):
                pltpu.async_copy(ones_v, acc_s.at[buf], add=True, sem=sem)

            def wait(sem):
                pltpu.make_async_copy(tab_hbm.at[pl.ds(0, W)], ones_v,
                                      sem).wait()

            load_idx(0, idx0_v)
            scat(idx0_v, sem0)

            def body(p, _):
                @pl.when(p > 0)
                def _():
                    wait(sem1)

                load_idx(2 * p + 1, idx1_v)
                scat(idx1_v, sem1)
                wait(sem0)

                @pl.when(p < NW // 2 - 1)
                def _():
                    load_idx(2 * p + 2, idx0_v)
                    scat(idx0_v, sem0)

                return 0

            lax.fori_loop(0, NW // 2, body, 0)
            wait(sem1)
            plsc.subcore_barrier()

            tabc = ones_v.at[pl.ds(0, ZR)]

            def epilogue(k, _):
                r0 = base_r + k * ZR
                pltpu.sync_copy(acc_s.at[pl.ds(r0, ZR)], chunk_v)
                pltpu.sync_copy(tab_hbm.at[pl.ds(r0, ZR)], tabc)

                def rows(i, _):
                    dis = _rsqrt_rows(chunk_v[i, :])
                    chunk_v[i, :] = dis
                    tabc[i, :] = tabc[i, :] * dis
                    return 0

                lax.fori_loop(0, ZR, rows, 0, unroll=8)
                pltpu.sync_copy(chunk_v, dis_hbm.at[pl.ds(r0, ZR)])
                pltpu.sync_copy(tabc, tabs_hbm.at[pl.ds(r0, ZR)])
                return 0

            lax.fori_loop(0, RPT // ZR, epilogue, 0)

        @pl.when(c == 0)
        def _():
            run(i_hbm, it_hbm, disi_hbm, items_hbm)

        @pl.when(c == 1)
        def _():
            run(u_hbm, ut_hbm, disu_hbm, users_hbm)

    return dis_kernel


def _make_agg_kernel(E, N, W, ZR):
    EPT = E // NS
    RPT = N // NS
    NW = EPT // W
    assert NW % 2 == 0 and EPT % W == 0 and RPT % ZR == 0 and ZR <= W

    @functools.partial(
        pl.kernel,
        out_type=(jax.ShapeDtypeStruct((N, D), jnp.float32),
                  jax.ShapeDtypeStruct((N, D), jnp.float32)),
        mesh=_mesh(),
        compiler_params=_SC_PARAMS,
        scratch_types=[
            pltpu.VMEM((W,), jnp.int32),
            pltpu.VMEM((W,), jnp.int32),
            pltpu.VMEM((W,), jnp.int32),
            pltpu.VMEM((W,), jnp.int32),
            pltpu.VMEM((W, D), jnp.float32),
            pltpu.VMEM((W, D), jnp.float32),
            pltpu.VMEM((ZR, D), jnp.float32),
            pltpu.VMEM_SHARED((N, D), jnp.float32),
            pltpu.SemaphoreType.DMA,
            pltpu.SemaphoreType.DMA,
            pltpu.SemaphoreType.DMA,
            pltpu.SemaphoreType.DMA,
        ],
    )
    def agg_kernel(u_hbm, i_hbm, users_hbm, items_hbm, disu_hbm, disi_hbm,
                   outu_hbm, outi_hbm,
                   dst0_v, src0_v, dst1_v, src1_v, rows0_v, rows1_v, zero_v,
                   acc_s, semg0, semg1, sems0, sems1):
        c = lax.axis_index("c")
        s = lax.axis_index("s")
        base_r = s * RPT

        def fill_zero(i, _):
            zero_v[i, :] = jnp.zeros((D,), jnp.float32)
            return 0

        lax.fori_loop(0, ZR, fill_zero, 0, unroll=8)

        def zero_acc(i, _):
            pltpu.sync_copy(zero_v, acc_s.at[pl.ds(base_r + i * ZR, ZR)])
            return 0

        lax.fori_loop(0, RPT // ZR, zero_acc, 0)
        plsc.subcore_barrier()

        def run(dst_hbm, src_hbm, tab_hbm, dis_hbm, out_hbm):
            base = s * EPT

            def load_idx(w, db, sb):
                pltpu.sync_copy(dst_hbm.at[pl.ds(base + w * W, W)], db)
                pltpu.sync_copy(src_hbm.at[pl.ds(base + w * W, W)], sb)

            def wait(sem, rows_v):
                pltpu.make_async_copy(tab_hbm.at[pl.ds(0, W)], rows_v,
                                      sem).wait()

            load_idx(0, dst0_v, src0_v)
            pltpu.async_copy(tab_hbm.at[src0_v], rows0_v, sem=semg0)

            def body(p, _):
                @pl.when(p > 0)
                def _():
                    wait(sems1, rows1_v)

                load_idx(2 * p + 1, dst1_v, src1_v)
                pltpu.async_copy(tab_hbm.at[src1_v], rows1_v, sem=semg1)
                wait(semg0, rows0_v)
                pltpu.async_copy(rows0_v, acc_s.at[dst0_v], add=True,
                                 sem=sems0)

                @pl.when(p < NW // 2 - 1)
                def _():
                    wait(sems0, rows0_v)
                    load_idx(2 * p + 2, dst0_v, src0_v)
                    pltpu.async_copy(tab_hbm.at[src0_v], rows0_v, sem=semg0)

                wait(semg1, rows1_v)
                pltpu.async_copy(rows1_v, acc_s.at[dst1_v], add=True,
                                 sem=sems1)
                return 0

            lax.fori_loop(0, NW // 2, body, 0)
            wait(sems0, rows0_v)
            wait(sems1, rows1_v)
            plsc.subcore_barrier()

            disc = rows0_v.at[pl.ds(0, ZR)]

            def epilogue(k, _):
                r0 = base_r + k * ZR
                pltpu.sync_copy(acc_s.at[pl.ds(r0, ZR)], zero_v)
                pltpu.sync_copy(dis_hbm.at[pl.ds(r0, ZR)], disc)

                def rows(i, _):
                    zero_v[i, :] = zero_v[i, :] * disc[i, :]
                    return 0

                lax.fori_loop(0, ZR, rows, 0, unroll=8)
                pltpu.sync_copy(zero_v, out_hbm.at[pl.ds(r0, ZR)])
                return 0

            lax.fori_loop(0, RPT // ZR, epilogue, 0)

        @pl.when(c == 0)
        def _():
            run(u_hbm, i_hbm, items_hbm, disu_hbm, outu_hbm)

        @pl.when(c == 1)
        def _():
            run(i_hbm, u_hbm, users_hbm, disi_hbm, outi_hbm)

    return agg_kernel


def kernel(u_row, i_col, user_table, item_table):
    E = u_row.shape[0]
    N = user_table.shape[0]
    W = 400
    ZR = 250

    dis_i, item_s, dis_u, user_s = _make_dis_kernel(E, N, W, ZR)(
        u_row, i_col, user_table, item_table)
    user_emb, item_emb = _make_agg_kernel(E, N, W, ZR)(
        u_row, i_col, user_s, item_s, dis_u, dis_i)
    return user_emb, item_emb
